# baseline (speedup 1.0000x reference)
import jax
import jax.numpy as jnp
from jax import lax
from jax.experimental import pallas as pl
from jax.experimental.pallas import tpu as pltpu
from jax.experimental.pallas import tpu_sc as plsc

B, H, W, C = 8, 256, 256, 32
N = 1000
ROI = 31
HALF = 15
NPOS = ROI * ROI

_INFO = plsc.get_sparse_core_info()
NC, NS = _INFO.num_cores, _INFO.num_subcores
NW = NC * NS
NBLK = 8
NLANE = 128
NG = NW // NBLK
STEP = 240
HALVES = 242


def _roi_body(map_hbm, b_hbm, cy_hbm, cx_hbm, out_hbm,
              bbase_v, cy_v, cx_v, xc_v, xok_v,
              idx_a, idx_b, msk_a, msk_b,
              gbuf_a, gbuf_b, tbuf_a, tbuf_b,
              gsem_a, gsem_b, wsem_a, wsem_b):
    wid = lax.axis_index("s") * NC + lax.axis_index("c")
    nb = wid % NBLK
    grp = wid // NBLK
    p0 = grp * STEP

    pltpu.sync_copy(cy_hbm.at[pl.ds(nb * NLANE, NLANE)], cy_v)
    pltpu.sync_copy(cx_hbm.at[pl.ds(nb * NLANE, NLANE)], cx_v)
    pltpu.sync_copy(b_hbm.at[pl.ds(nb * NLANE, NLANE)], bbase_v)

    lane = lax.iota(jnp.int32, 16)

    def setup_n(m, _):
        sl = pl.ds(m * 16, 16)
        bbase_v[sl] = bbase_v[sl] * (H * W)
        return 0

    lax.fori_loop(0, 8, setup_n, 0)

    def setup_x(c, _):
        def inner(m, _):
            sl = pl.ds(m * 16, 16)
            x = cx_v[sl] - HALF + c
            xc_v[c, sl] = jnp.clip(x, 0, W - 1)
            xok_v[c, sl] = jnp.where(
                (x >= 0) & (x < W), 1.0, 0.0).astype(jnp.float32)
            return 0
        lax.fori_loop(0, 8, inner, 0)
        return 0

    lax.fori_loop(0, ROI, setup_x, 0)

    def pos_rc(i):
        p = jnp.minimum(p0 + i, NPOS - 1)
        return p // ROI, p % ROI

    def compute_pos(i, idx_v, msk_v):
        r, c = pos_rc(i)
        for m in range(8):
            sl = pl.ds(m * 16, 16)
            y = cy_v[sl] - HALF + r
            yokf = jnp.where((y >= 0) & (y < H), 1.0, 0.0).astype(jnp.float32)
            rowb = bbase_v[sl] + jnp.clip(y, 0, H - 1) * W
            idx_v[sl] = rowb + xc_v[c, sl]
            msk_v[sl] = yokf * xok_v[c, sl]

    def fire_gather(idx_v, gbuf_v, gsem):
        pltpu.async_copy(map_hbm.at[idx_v], gbuf_v, gsem)

    def drain_write(tbuf_v, wsem):
        for chb in range(4):
            pltpu.make_async_copy(
                tbuf_v.at[chb], out_hbm.at[0, 0, chb, 0], wsem).wait()

    def half(i, slot):
        idx_v, msk_v, gbuf_v, tbuf_v, gsem, wsem = slot
        pltpu.make_async_copy(map_hbm.at[idx_v], gbuf_v, gsem).wait()

        @pl.when(i >= 2)
        def _():
            drain_write(tbuf_v, wsem)

        for m in range(8):
            n16 = lane + m * 16
            mk = msk_v[pl.ds(m * 16, 16)]
            for ch in range(C):
                v = plsc.load_gather(
                    gbuf_v, [n16, jnp.full((16,), ch, jnp.int32)])
                tbuf_v[ch // 8, ch % 8, pl.ds(m * 16, 16)] = v * mk

        r, c = pos_rc(i)
        for chb in range(4):
            pltpu.async_copy(tbuf_v.at[chb], out_hbm.at[r, c, chb, nb], wsem)

        compute_pos(i + 2, idx_v, msk_v)
        fire_gather(idx_v, gbuf_v, gsem)

    slots = ((idx_a, msk_a, gbuf_a, tbuf_a, gsem_a, wsem_a),
             (idx_b, msk_b, gbuf_b, tbuf_b, gsem_b, wsem_b))

    compute_pos(0, idx_a, msk_a)
    fire_gather(idx_a, gbuf_a, gsem_a)
    compute_pos(1, idx_b, msk_b)
    fire_gather(idx_b, gbuf_b, gsem_b)

    def pair(i2, _):
        half(2 * i2, slots[0])
        half(2 * i2 + 1, slots[1])
        return 0

    lax.fori_loop(0, HALVES // 2, pair, 0)

    for slot in slots:
        idx_v, msk_v, gbuf_v, tbuf_v, gsem, wsem = slot
        pltpu.make_async_copy(map_hbm.at[idx_v], gbuf_v, gsem).wait()
        drain_write(tbuf_v, wsem)


@jax.jit
def kernel(encoded_poses, roi_centers):
    flat_map = encoded_poses.reshape(B * H * W, C)
    rc_pad = jnp.concatenate(
        [roi_centers, jnp.zeros((NBLK * NLANE - N, 3), jnp.int32)], axis=0)
    b_arr = rc_pad[:, 0]
    cy_arr = rc_pad[:, 1]
    cx_arr = rc_pad[:, 2]

    mesh = plsc.VectorSubcoreMesh(core_axis_name="c", subcore_axis_name="s")
    run = pl.kernel(
        _roi_body,
        out_type=jax.ShapeDtypeStruct((ROI, ROI, 4, NBLK, 8, NLANE),
                                      jnp.float32),
        mesh=mesh,
        compiler_params=pltpu.CompilerParams(
            use_tc_tiling_on_sc=False, needs_layout_passes=False),
        scratch_types=[
            pltpu.VMEM((NLANE,), jnp.int32),
            pltpu.VMEM((NLANE,), jnp.int32),
            pltpu.VMEM((NLANE,), jnp.int32),
            pltpu.VMEM((ROI, NLANE), jnp.int32),
            pltpu.VMEM((ROI, NLANE), jnp.float32),
            pltpu.VMEM((NLANE,), jnp.int32),
            pltpu.VMEM((NLANE,), jnp.int32),
            pltpu.VMEM((NLANE,), jnp.float32),
            pltpu.VMEM((NLANE,), jnp.float32),
            pltpu.VMEM((NLANE, C), jnp.float32),
            pltpu.VMEM((NLANE, C), jnp.float32),
            pltpu.VMEM((4, 8, NLANE), jnp.float32),
            pltpu.VMEM((4, 8, NLANE), jnp.float32),
            pltpu.SemaphoreType.DMA,
            pltpu.SemaphoreType.DMA,
            pltpu.SemaphoreType.DMA,
            pltpu.SemaphoreType.DMA,
        ],
    )
    out6 = run(flat_map, b_arr, cy_arr, cx_arr)
    t = out6.transpose(3, 5, 0, 1, 2, 4)
    return t.reshape(NBLK * NLANE, ROI, ROI, C)[:N]
